# Initial kernel scaffold; baseline (speedup 1.0000x reference)
import jax
import jax.numpy as jnp
from jax.experimental import pallas as pl


def kernel(x, rand, w_perc, w1, b1, w2, b2):
    raise NotImplementedError("write your pallas kernel here")


# speedup vs baseline: 3.6540x; 3.6540x over previous
import jax
import jax.numpy as jnp
from jax import lax
from jax.experimental import pallas as pl
from jax.experimental.pallas import tpu as pltpu

_C = 16
_H = 256
_W = 256
_N = _H * _W
_CHUNK = 8192
_FIRE_RATE = 0.5
_ALIVE = 0.1
_NEG = -3.0e38


def _roll(a, s):
    return pltpu.roll(a, s % a.shape[1], axis=1)


def _nca_body(x_ref, rand_ref, wA_ref, wB_ref, out_ref):
    xv = x_ref[0]
    lane = lax.broadcasted_iota(jnp.int32, (_C, _N), 1)
    col = jnp.bitwise_and(lane, _W - 1)
    m_c0 = col == 0
    m_cL = col == _W - 1

    xp1 = jnp.where(m_cL, _roll(xv, _W - 1),
                    _roll(xv, -1))
    xm1 = jnp.where(m_c0, _roll(xv, -(_W - 1)),
                    _roll(xv, 1))

    ones = jnp.ones((1, _CHUNK), jnp.float32)
    bases = (xm1, xv, xp1)

    for j in range(_N // _CHUNK):
        sl = slice(j * _CHUNK, (j + 1) * _CHUNK)
        pieces = []
        for dr in (-1, 0, 1):
            for b in bases:
                pieces.append(_roll(b, -_W * dr)[:, sl])
        p = jnp.concatenate(pieces + [ones], axis=0)
        h = jnp.dot(wA_ref[...], p, preferred_element_type=jnp.float32)
        h = jnp.maximum(h, 0.0)
        h = jnp.concatenate([h, ones], axis=0)
        d = jnp.dot(wB_ref[...], h, preferred_element_type=jnp.float32)
        fire = rand_ref[0, :, sl] <= _FIRE_RATE
        out_ref[0, :, sl] = xv[:, sl] + jnp.where(fire, d, 0.0)

    lane1 = lax.broadcasted_iota(jnp.int32, (1, _N), 1)
    col1 = jnp.bitwise_and(lane1, _W - 1)
    m0 = col1 == 0
    mL = col1 == _W - 1
    m_top = lane1 < _W
    m_bot = lane1 >= _N - _W

    def pool(a):
        up = jnp.where(m_top, _NEG, _roll(a, _W))
        dn = jnp.where(m_bot, _NEG, _roll(a, -_W))
        rm = jnp.maximum(jnp.maximum(up, dn), a)
        le = jnp.where(mL, _NEG, _roll(rm, -1))
        ri = jnp.where(m0, _NEG, _roll(rm, 1))
        return jnp.maximum(jnp.maximum(le, ri), rm)

    alive_pre = pool(xv[3:4, :]) > _ALIVE
    alive_post = pool(out_ref[0, 3:4, :]) > _ALIVE
    life = alive_pre & alive_post
    out_ref[0] = jnp.where(life, out_ref[0, :, :], 0.0)


def kernel(x, rand, w_perc, w1, b1, w2, b2):
    B, C, H, W = x.shape
    N = H * W
    f32 = jnp.float32
    wp = w_perc.reshape(C, 3, 3, 3)
    w1r = w1.reshape(128, C, 3)
    W2 = jnp.einsum('ock,ckij->oijc', w1r, wp).reshape(128, 9 * C)
    wA = jnp.concatenate([W2, b1[:, None]], axis=1).astype(f32)
    wB = jnp.concatenate([w2, b2[:, None]], axis=1).astype(f32)

    xf = x.reshape(B, C, N)
    rf = rand.reshape(B, 1, N)
    out = pl.pallas_call(
        _nca_body,
        grid=(B,),
        in_specs=[
            pl.BlockSpec((1, C, N), lambda b: (b, 0, 0)),
            pl.BlockSpec((1, 1, N), lambda b: (b, 0, 0)),
            pl.BlockSpec((128, 9 * C + 1), lambda b: (0, 0)),
            pl.BlockSpec((C, 129), lambda b: (0, 0)),
        ],
        out_specs=pl.BlockSpec((1, C, N), lambda b: (b, 0, 0)),
        out_shape=jax.ShapeDtypeStruct((B, C, N), f32),
        compiler_params=pltpu.CompilerParams(
            dimension_semantics=("parallel",),
            vmem_limit_bytes=56 * 1024 * 1024,
        ),
    )(xf, rf, wA, wB)
    return out.reshape(B, C, H, W)
